# Initial kernel scaffold; baseline (speedup 1.0000x reference)
import jax
import jax.numpy as jnp
from jax.experimental import pallas as pl


def kernel(x, pos_embedding):
    raise NotImplementedError("write your pallas kernel here")


# speedup vs baseline: 4.5837x; 4.5837x over previous
import jax
import jax.numpy as jnp
from jax.experimental import pallas as pl


def _add_pos_kernel(x_ref, pe_ref, out_ref):
    pe = pe_ref[...]
    out_ref[...] = x_ref[...] + pe[None, :, None, :]


def kernel(x, pos_embedding):
    B, T, S, D = x.shape
    TB = 256
    grid = (B, T // TB)
    return pl.pallas_call(
        _add_pos_kernel,
        grid=grid,
        in_specs=[
            pl.BlockSpec((1, TB, S, D), lambda b, t: (b, t, 0, 0)),
            pl.BlockSpec((TB, D), lambda b, t: (t, 0)),
        ],
        out_specs=pl.BlockSpec((1, TB, S, D), lambda b, t: (b, t, 0, 0)),
        out_shape=jax.ShapeDtypeStruct((B, T, S, D), x.dtype),
    )(x, pos_embedding)


# speedup vs baseline: 4.6887x; 1.0229x over previous
import jax
import jax.numpy as jnp
from jax.experimental import pallas as pl


def _add_pos_kernel(x_ref, pe_ref, out_ref):
    pe = pe_ref[...]
    out_ref[...] = x_ref[...] + pe[None, :, None, :]


def kernel(x, pos_embedding):
    B, T, S, D = x.shape
    TB = 512
    grid = (B, T // TB)
    return pl.pallas_call(
        _add_pos_kernel,
        grid=grid,
        in_specs=[
            pl.BlockSpec((1, TB, S, D), lambda b, t: (b, t, 0, 0)),
            pl.BlockSpec((TB, D), lambda b, t: (t, 0)),
        ],
        out_specs=pl.BlockSpec((1, TB, S, D), lambda b, t: (b, t, 0, 0)),
        out_shape=jax.ShapeDtypeStruct((B, T, S, D), x.dtype),
    )(x, pos_embedding)


# speedup vs baseline: 5.0232x; 1.0713x over previous
import jax
import jax.numpy as jnp
from jax.experimental import pallas as pl


def _add_pos_kernel(x_ref, pe_ref, out_ref):
    pe = pe_ref[...]
    out_ref[...] = x_ref[...] + pe[None, :, None, :]


def kernel(x, pos_embedding):
    B, T, S, D = x.shape
    TB = 512
    grid = (T // TB, B)
    return pl.pallas_call(
        _add_pos_kernel,
        grid=grid,
        in_specs=[
            pl.BlockSpec((1, TB, S, D), lambda t, b: (b, t, 0, 0)),
            pl.BlockSpec((TB, D), lambda t, b: (t, 0)),
        ],
        out_specs=pl.BlockSpec((1, TB, S, D), lambda t, b: (b, t, 0, 0)),
        out_shape=jax.ShapeDtypeStruct((B, T, S, D), x.dtype),
    )(x, pos_embedding)
